# baseline (speedup 1.0000x reference)
import functools

import jax
import jax.numpy as jnp
from jax import lax
from jax.experimental import pallas as pl
from jax.experimental.pallas import tpu as pltpu
from jax.experimental.pallas import tpu_sc as plsc

B = 16
R = 64
C = 256
N = 32768
HW = R * R
CHW = C // 2
SUB = 128
NSUB = N // SUB
NTILE = 16
KPT = NSUB // NTILE
ROWS_PT = HW // NTILE
NBUF = 2
PHASES = B * 2


def _sc_body(feats_hbm, zeros_hbm, bidx_hbm, sidx_hbm, out_hbm,
             fbuf, zbuf, bbuf, sbuf, ibuf,
             acc0, acc1, sem_w, sem_z0, sem_z1):
    accs = (acc0, acc1)
    sem_z = (sem_z0, sem_z1)
    tid = lax.axis_index("s")
    r0 = tid * ROWS_PT

    pltpu.sync_copy(zeros_hbm, zbuf)
    for k in range(KPT):
        chunk = k * NTILE + tid
        pltpu.sync_copy(bidx_hbm.at[pl.ds(chunk * SUB, SUB)],
                        bbuf.at[pl.ds(k * SUB, SUB)])
        pltpu.sync_copy(sidx_hbm.at[pl.ds(chunk * SUB, SUB)],
                        sbuf.at[pl.ds(k * SUB, SUB)])
    for w in range(NBUF):
        pltpu.sync_copy(zbuf, accs[w].at[pl.ds(r0, ROWS_PT)])

    desc_w = [None] * NBUF
    desc_z = [None] * NBUF

    for i in range(PHASES):
        w = i % NBUF
        b = i // 2
        c0 = (i % 2) * CHW
        acc = accs[w]

        if desc_z[w] is not None:
            desc_z[w].wait()
            desc_z[w] = None
        plsc.subcore_barrier()

        def sub_body(k, carry, acc=acc, c0=c0, b=b):
            base = k * SUB
            bfirst = bbuf[pl.ds(base, 16)][0]
            blast = bbuf[pl.ds(base + SUB - 16, 16)][15]

            @pl.when(jnp.logical_and(bfirst <= b, b <= blast))
            def _():
                chunk = k * NTILE + tid
                pltpu.sync_copy(
                    feats_hbm.at[pl.ds(chunk * SUB, SUB), pl.ds(c0, CHW)],
                    fbuf)
                for j in range(SUB // 16):
                    vb = bbuf[pl.ds(base + j * 16, 16)]
                    vs = sbuf[pl.ds(base + j * 16, 16)]
                    ibuf[pl.ds(j * 16, 16)] = jnp.where(vb == b, vs, HW + tid)
                pltpu.sync_copy(fbuf, acc.at[ibuf], add=True)

            return carry

        lax.fori_loop(0, KPT, sub_body, 0)

        if i >= 1:
            wp = (i - 1) % NBUF
            desc_w[wp].wait()
            desc_w[wp] = None
            if i - 1 + NBUF < PHASES:
                desc_z[wp] = pltpu.async_copy(
                    zbuf, accs[wp].at[pl.ds(r0, ROWS_PT)], sem_z[wp])

        plsc.subcore_barrier()
        desc_w[w] = pltpu.async_copy(
            acc.at[pl.ds(r0, ROWS_PT)],
            out_hbm.at[b, pl.ds(r0, ROWS_PT), pl.ds(c0, CHW)],
            sem_w)

    for d in desc_w + desc_z:
        if d is not None:
            d.wait()


_scatter_sc = functools.partial(
    pl.kernel,
    out_type=jax.ShapeDtypeStruct((B, HW, C), jnp.float32),
    mesh=plsc.VectorSubcoreMesh(core_axis_name="c", subcore_axis_name="s",
                                num_cores=1),
    scratch_types=[
        pltpu.VMEM((SUB, CHW), jnp.float32),
        pltpu.VMEM((ROWS_PT, CHW), jnp.float32),
        pltpu.VMEM((KPT * SUB,), jnp.int32),
        pltpu.VMEM((KPT * SUB,), jnp.int32),
        pltpu.VMEM((SUB,), jnp.int32),
        pltpu.VMEM_SHARED((HW + NTILE, CHW), jnp.float32),
        pltpu.VMEM_SHARED((HW + NTILE, CHW), jnp.float32),
        pltpu.SemaphoreType.DMA,
        pltpu.SemaphoreType.DMA,
        pltpu.SemaphoreType.DMA,
    ],
)(_sc_body)


TH = 512


def _t_body(in_ref, out_ref):
    out_ref[0] = in_ref[0].T


_transpose_tc = pl.pallas_call(
    _t_body,
    grid=(B, HW // TH),
    in_specs=[pl.BlockSpec((1, TH, C), lambda b, j: (b, j, 0))],
    out_specs=pl.BlockSpec((1, C, TH), lambda b, j: (b, 0, j)),
    out_shape=jax.ShapeDtypeStruct((B, C, HW), jnp.float32),
)


@jax.jit
def kernel(feats, batch_idx, spatial_idx):
    bidx = batch_idx.astype(jnp.int32)
    sidx = spatial_idx.astype(jnp.int32)
    zeros = jnp.zeros((ROWS_PT, CHW), jnp.float32)
    dense = _scatter_sc(feats, zeros, bidx, sidx)
    out = _transpose_tc(dense)
    return out.reshape(B, C, R, R)


# speedup vs baseline: 1.0585x; 1.0585x over previous
import functools

import jax
import jax.numpy as jnp
from jax import lax
from jax.experimental import pallas as pl
from jax.experimental.pallas import tpu as pltpu
from jax.experimental.pallas import tpu_sc as plsc

B = 16
R = 64
C = 256
N = 32768
HW = R * R
CHW = C // 2
SUB = 128
NSUB = N // SUB
NTILE = 16
KPT = NSUB // NTILE
ROWS_PT = HW // NTILE


def _sc_body(feats_hbm, zeros_hbm, bidx_hbm, sidx_hbm, out_hbm,
             fbuf, bbuf, sbuf, ibuf, acc0, acc1, sem_w0, sem_w1):
    accs = (acc0, acc1)
    sems = (sem_w0, sem_w1)
    tid = lax.axis_index("s")
    r0 = tid * ROWS_PT

    for k in range(KPT):
        chunk = k * NTILE + tid
        pltpu.sync_copy(bidx_hbm.at[pl.ds(chunk * SUB, SUB)],
                        bbuf.at[pl.ds(k * SUB, SUB)])
        pltpu.sync_copy(sidx_hbm.at[pl.ds(chunk * SUB, SUB)],
                        sbuf.at[pl.ds(k * SUB, SUB)])
    for w in range(2):
        pltpu.sync_copy(zeros_hbm, accs[w].at[pl.ds(r0, ROWS_PT)])

    desc_w = [None, None]
    for i in range(2 * B):
        w = i % 2
        b = i // 2
        c0 = w * CHW
        acc = accs[w]

        if desc_w[w] is not None:
            desc_w[w].wait()
            desc_w[w] = None
        plsc.subcore_barrier()

        def sub_body(k, carry, acc=acc, c0=c0, b=b):
            base = k * SUB
            bfirst = bbuf[pl.ds(base, 16)][0]
            blast = bbuf[pl.ds(base + SUB - 16, 16)][15]

            @pl.when(jnp.logical_and(bfirst <= b, b <= blast))
            def _():
                chunk = k * NTILE + tid
                pltpu.sync_copy(
                    feats_hbm.at[pl.ds(chunk * SUB, SUB), pl.ds(c0, CHW)],
                    fbuf)
                for j in range(SUB // 16):
                    vb = bbuf[pl.ds(base + j * 16, 16)]
                    vs = sbuf[pl.ds(base + j * 16, 16)]
                    ibuf[pl.ds(j * 16, 16)] = jnp.where(vb == b, vs, HW + tid)
                pltpu.sync_copy(fbuf, acc.at[ibuf], add=True)

            return carry

        lax.fori_loop(0, KPT, sub_body, 0)
        plsc.subcore_barrier()
        desc_w[w] = pltpu.async_copy(
            acc.at[pl.ds(r0, ROWS_PT)],
            out_hbm.at[b, pl.ds(r0, ROWS_PT), pl.ds(c0, CHW)],
            sems[w])

    for d in desc_w:
        if d is not None:
            d.wait()


_scatter_sc = functools.partial(
    pl.kernel,
    out_type=jax.ShapeDtypeStruct((B, HW, C), jnp.float32),
    mesh=plsc.VectorSubcoreMesh(core_axis_name="c", subcore_axis_name="s",
                                num_cores=1),
    scratch_types=[
        pltpu.VMEM((SUB, CHW), jnp.float32),
        pltpu.VMEM((KPT * SUB,), jnp.int32),
        pltpu.VMEM((KPT * SUB,), jnp.int32),
        pltpu.VMEM((SUB,), jnp.int32),
        pltpu.VMEM_SHARED((HW + NTILE, CHW), jnp.float32),
        pltpu.VMEM_SHARED((HW + NTILE, CHW), jnp.float32),
        pltpu.SemaphoreType.DMA,
        pltpu.SemaphoreType.DMA,
    ],
)(_sc_body)


TH = 512


def _t_body(cur_ref, prev_ref, out_ref):
    bb = pl.program_id(0)
    prev = jnp.where(bb == 0, jnp.zeros_like(prev_ref[0]), prev_ref[0])
    out_ref[0] = (cur_ref[0] - prev).T


_transpose_tc = pl.pallas_call(
    _t_body,
    grid=(B, HW // TH),
    in_specs=[pl.BlockSpec((1, TH, C), lambda b, j: (b, j, 0)),
              pl.BlockSpec((1, TH, C),
                           lambda b, j: (jnp.maximum(b, 1) - 1, j, 0))],
    out_specs=pl.BlockSpec((1, C, TH), lambda b, j: (b, 0, j)),
    out_shape=jax.ShapeDtypeStruct((B, C, HW), jnp.float32),
)


@jax.jit
def kernel(feats, batch_idx, spatial_idx):
    bidx = batch_idx.astype(jnp.int32)
    sidx = spatial_idx.astype(jnp.int32)
    zeros = jnp.zeros((ROWS_PT, CHW), jnp.float32)
    dense = _scatter_sc(feats, zeros, bidx, sidx)
    out = _transpose_tc(dense, dense)
    return out.reshape(B, C, R, R)


# speedup vs baseline: 1.1699x; 1.1052x over previous
import functools

import jax
import jax.numpy as jnp
from jax import lax
from jax.experimental import pallas as pl
from jax.experimental.pallas import tpu as pltpu
from jax.experimental.pallas import tpu_sc as plsc

B = 16
R = 64
C = 256
N = 32768
HW = R * R
CHW = C // 2
SUB = 128
NSUB = N // SUB
NTILE = 16
KPT = NSUB // NTILE
ROWS_PT = HW // NTILE


def _sc_body(feats_hbm, zeros_hbm, comb_hbm, out_hbm,
             fbuf, cbuf, ibuf, acc0, acc1, sem_w0, sem_w1, sem_p):
    accs = (acc0, acc1)
    sems = (sem_w0, sem_w1)
    tid = lax.axis_index("s")
    r0 = tid * ROWS_PT

    pend = []
    for k in range(KPT):
        chunk = k * NTILE + tid
        pend.append(pltpu.async_copy(comb_hbm.at[pl.ds(chunk * SUB, SUB)],
                                     cbuf.at[pl.ds(k * SUB, SUB)], sem_p))
    for w in range(2):
        pend.append(pltpu.async_copy(zeros_hbm,
                                     accs[w].at[pl.ds(r0, ROWS_PT)], sems[w]))
    for d in pend:
        d.wait()

    desc_w = [None, None]
    for i in range(2 * B):
        w = i % 2
        b = i // 2
        c0 = w * CHW
        acc = accs[w]

        if desc_w[w] is not None:
            desc_w[w].wait()
            desc_w[w] = None
        plsc.subcore_barrier()

        def sub_body(k, carry, acc=acc, c0=c0, b=b):
            base = k * SUB
            bfirst = cbuf[pl.ds(base, 16)][0] >> 16
            blast = cbuf[pl.ds(base + SUB - 16, 16)][15] >> 16

            @pl.when(jnp.logical_and(bfirst <= b, b <= blast))
            def _():
                chunk = k * NTILE + tid
                pltpu.sync_copy(
                    feats_hbm.at[pl.ds(chunk * SUB, SUB), pl.ds(c0, CHW)],
                    fbuf)
                for j in range(SUB // 16):
                    vc = cbuf[pl.ds(base + j * 16, 16)]
                    ibuf[pl.ds(j * 16, 16)] = jnp.where(
                        (vc >> 16) == b, vc & 0xFFFF, HW + tid)
                pltpu.sync_copy(fbuf, acc.at[ibuf], add=True)

            return carry

        lax.fori_loop(0, KPT, sub_body, 0)
        plsc.subcore_barrier()
        desc_w[w] = pltpu.async_copy(
            acc.at[pl.ds(r0, ROWS_PT)],
            out_hbm.at[b, pl.ds(r0, ROWS_PT), pl.ds(c0, CHW)],
            sems[w])

    for d in desc_w:
        if d is not None:
            d.wait()


_scatter_sc = functools.partial(
    pl.kernel,
    out_type=jax.ShapeDtypeStruct((B, HW, C), jnp.float32),
    mesh=plsc.VectorSubcoreMesh(core_axis_name="c", subcore_axis_name="s",
                                num_cores=1),
    scratch_types=[
        pltpu.VMEM((SUB, CHW), jnp.float32),
        pltpu.VMEM((KPT * SUB,), jnp.int32),
        pltpu.VMEM((SUB,), jnp.int32),
        pltpu.VMEM_SHARED((HW + NTILE, CHW), jnp.float32),
        pltpu.VMEM_SHARED((HW + NTILE, CHW), jnp.float32),
        pltpu.SemaphoreType.DMA,
        pltpu.SemaphoreType.DMA,
        pltpu.SemaphoreType.DMA,
    ],
)(_sc_body)


TH = 512


def _t_body(cur_ref, out_ref, prev_scr):
    bb = pl.program_id(1)
    cur = cur_ref[0]
    prev = jnp.where(bb == 0, jnp.zeros_like(cur), prev_scr[...])
    out_ref[0] = (cur - prev).T
    prev_scr[...] = cur


_transpose_tc = pl.pallas_call(
    _t_body,
    grid=(HW // TH, B),
    in_specs=[pl.BlockSpec((1, TH, C), lambda j, b: (b, j, 0))],
    out_specs=pl.BlockSpec((1, C, TH), lambda j, b: (b, 0, j)),
    out_shape=jax.ShapeDtypeStruct((B, C, HW), jnp.float32),
    scratch_shapes=[pltpu.VMEM((TH, C), jnp.float32)],
)


@jax.jit
def kernel(feats, batch_idx, spatial_idx):
    comb = ((batch_idx.astype(jnp.int32) << 16)
            | spatial_idx.astype(jnp.int32))
    zeros = jnp.zeros((ROWS_PT, CHW), jnp.float32)
    dense = _scatter_sc(feats, zeros, comb)
    out = _transpose_tc(dense)
    return out.reshape(B, C, R, R)
